# Initial kernel scaffold; baseline (speedup 1.0000x reference)
import jax
import jax.numpy as jnp
from jax.experimental import pallas as pl


def kernel(x):
    raise NotImplementedError("write your pallas kernel here")


# speedup vs baseline: 1.2928x; 1.2928x over previous
import jax
import jax.numpy as jnp
from jax.experimental import pallas as pl
from jax.experimental.pallas import tpu as pltpu

_KS = 3
_ST = 2
_H = 512
_W = 512
_HO = (_H - _KS) // _ST + 1
_WO = (_W - _KS) // _ST + 1
_LC = 128
_NC = _W // _LC


def _pool_body(x_ref, t_ref, o_ref, acc_ref):
    c = pl.program_id(1)
    a = x_ref[:, pl.ds(0, _HO, _ST), :]
    b = x_ref[:, pl.ds(1, _HO, _ST), :]
    d = x_ref[:, pl.ds(2, _HO, _ST), :]
    scale = jnp.float32(1.0 / (_KS * _KS))
    rows = ((a + b + d)[0] * scale).astype(jnp.bfloat16)
    part = jnp.dot(rows, t_ref[0], preferred_element_type=jnp.float32)

    @pl.when(c == 0)
    def _():
        acc_ref[...] = part

    @pl.when(c != 0)
    def _():
        acc_ref[...] += part

    @pl.when(c == _NC - 1)
    def _():
        o_ref[0] = acc_ref[:, :_WO]


def _colpool_matrix():
    k = jnp.arange(_W, dtype=jnp.int32)[:, None]
    j = jnp.arange(256, dtype=jnp.int32)[None, :]
    d = k - _ST * j
    t = ((d >= 0) & (d < _KS)).astype(jnp.bfloat16)
    return t.reshape(_NC, _LC, 256)


def kernel(x):
    bsz, ch, h, w = x.shape
    n = bsz * ch
    xr = x.reshape(n, h, w)
    tmat = _colpool_matrix()
    out = pl.pallas_call(
        _pool_body,
        grid=(n, _NC),
        in_specs=[
            pl.BlockSpec((1, _H, _LC), lambda i, c: (i, 0, c)),
            pl.BlockSpec((1, _LC, 256), lambda i, c: (c, 0, 0)),
        ],
        out_specs=pl.BlockSpec((1, _HO, _WO), lambda i, c: (i, 0, 0)),
        out_shape=jax.ShapeDtypeStruct((n, _HO, _WO), x.dtype),
        scratch_shapes=[pltpu.VMEM((_HO, 256), jnp.float32)],
        compiler_params=pltpu.CompilerParams(
            dimension_semantics=("parallel", "arbitrary"),
        ),
    )(xr, tmat)
    return out.reshape(bsz, ch, _HO, _WO)


# speedup vs baseline: 6.0058x; 4.6455x over previous
import jax
import jax.numpy as jnp
from jax.experimental import pallas as pl
from jax.experimental.pallas import tpu as pltpu

_KS = 3
_ST = 2
_H = 512
_W = 512
_HO = (_H - _KS) // _ST + 1
_WO = (_W - _KS) // _ST + 1
_LC = 128
_NC = _W // _LC
_CG = 8


def _pool_body(x0_ref, x1_ref, x2_ref, x3_ref, t_ref, o_ref):
    slabs = (x0_ref, x1_ref, x2_ref, x3_ref)
    for g in range(_CG):
        acc = None
        for ci, xc in enumerate(slabs):
            a = xc[:, pl.ds(g, 1), pl.ds(0, _HO, _ST), :]
            b = xc[:, pl.ds(g, 1), pl.ds(1, _HO, _ST), :]
            d = xc[:, pl.ds(g, 1), pl.ds(2, _HO, _ST), :]
            rows = (a + b + d)[0, 0].astype(jnp.bfloat16)
            part = jnp.dot(rows, t_ref[ci],
                           preferred_element_type=jnp.float32)
            acc = part if acc is None else acc + part
        o_ref[0, g] = acc[:, :_WO]


def _colpool_matrix():
    k = jnp.arange(_W, dtype=jnp.int32)[:, None]
    j = jnp.arange(256, dtype=jnp.int32)[None, :]
    d = k - _ST * j
    t = jnp.where((d >= 0) & (d < _KS), 1.0 / (_KS * _KS), 0.0)
    return t.astype(jnp.bfloat16).reshape(_NC, _LC, 256)


def kernel(x):
    bsz, ch, h, w = x.shape
    tmat = _colpool_matrix()

    def _x_spec(ci):
        return pl.BlockSpec((1, _CG, _H, _LC), lambda b, g: (b, g, 0, ci))

    out = pl.pallas_call(
        _pool_body,
        grid=(bsz, ch // _CG),
        in_specs=[_x_spec(0), _x_spec(1), _x_spec(2), _x_spec(3),
                  pl.BlockSpec((_NC, _LC, 256), lambda b, g: (0, 0, 0))],
        out_specs=pl.BlockSpec((1, _CG, _HO, _WO), lambda b, g: (b, g, 0, 0)),
        out_shape=jax.ShapeDtypeStruct((bsz, ch, _HO, _WO), x.dtype),
        compiler_params=pltpu.CompilerParams(
            dimension_semantics=("parallel", "arbitrary"),
        ),
    )(x, x, x, x, tmat)
    return out


# speedup vs baseline: 6.0212x; 1.0026x over previous
import jax
import jax.numpy as jnp
from jax.experimental import pallas as pl
from jax.experimental.pallas import tpu as pltpu

_KS = 3
_ST = 2
_H = 512
_W = 512
_HO = (_H - _KS) // _ST + 1
_WO = (_W - _KS) // _ST + 1
_LC = 128
_NC = _W // _LC
_CG = 16


def _pool_body(x0_ref, x1_ref, x2_ref, x3_ref, t_ref, o_ref):
    slabs = (x0_ref, x1_ref, x2_ref, x3_ref)
    for g in range(_CG):
        acc = None
        for ci, xc in enumerate(slabs):
            a = xc[:, pl.ds(g, 1), pl.ds(0, _HO, _ST), :]
            b = xc[:, pl.ds(g, 1), pl.ds(1, _HO, _ST), :]
            d = xc[:, pl.ds(g, 1), pl.ds(2, _HO, _ST), :]
            rows = (a + b + d)[0, 0].astype(jnp.bfloat16)
            part = jnp.dot(rows, t_ref[ci],
                           preferred_element_type=jnp.float32)
            acc = part if acc is None else acc + part
        o_ref[0, g] = acc[:, :_WO]


def _colpool_matrix():
    k = jnp.arange(_W, dtype=jnp.int32)[:, None]
    j = jnp.arange(256, dtype=jnp.int32)[None, :]
    d = k - _ST * j
    t = jnp.where((d >= 0) & (d < _KS), 1.0 / (_KS * _KS), 0.0)
    return t.astype(jnp.bfloat16).reshape(_NC, _LC, 256)


def kernel(x):
    bsz, ch, h, w = x.shape
    tmat = _colpool_matrix()

    def _x_spec(ci):
        return pl.BlockSpec((1, _CG, _H, _LC), lambda b, g: (b, g, 0, ci))

    out = pl.pallas_call(
        _pool_body,
        grid=(bsz, ch // _CG),
        in_specs=[_x_spec(0), _x_spec(1), _x_spec(2), _x_spec(3),
                  pl.BlockSpec((_NC, _LC, 256), lambda b, g: (0, 0, 0))],
        out_specs=pl.BlockSpec((1, _CG, _HO, _WO), lambda b, g: (b, g, 0, 0)),
        out_shape=jax.ShapeDtypeStruct((bsz, ch, _HO, _WO), x.dtype),
        compiler_params=pltpu.CompilerParams(
            dimension_semantics=("parallel", "arbitrary"),
        ),
    )(x, x, x, x, tmat)
    return out
